# baseline (speedup 1.0000x reference)
import functools

import jax
import jax.numpy as jnp
from jax import lax
from jax.experimental import pallas as pl
from jax.experimental.pallas import tpu as pltpu
from jax.experimental.pallas import tpu_sc as plsc

N = 10000
NP = 10240
D = 128
E = 320000
NC = 2
NS = 16
NW = NC * NS
L = 16
EPW = E // NW
K = 16
CH = EPW // K
RPS = NP // NS


_TCB = 2048


def _tc_first_body(x_ref, w_ref, as_ref, ad_ref, h_ref, s_ref, d_ref):
    h = jnp.dot(x_ref[...], w_ref[...], preferred_element_type=jnp.float32)
    h_ref[...] = h
    s_ref[...] = jnp.sum(h * as_ref[...][None, :], axis=1)
    d_ref[...] = jnp.sum(h * ad_ref[...][None, :], axis=1)


def _tc_first(x_p, W, a_s, a_d):
    return pl.pallas_call(
        _tc_first_body,
        grid=(NP // _TCB,),
        in_specs=[
            pl.BlockSpec((_TCB, D), lambda i: (i, 0)),
            pl.BlockSpec((D, D), lambda i: (0, 0)),
            pl.BlockSpec((D,), lambda i: (0,)),
            pl.BlockSpec((D,), lambda i: (0,)),
        ],
        out_specs=[
            pl.BlockSpec((_TCB, D), lambda i: (i, 0)),
            pl.BlockSpec((_TCB,), lambda i: (i,)),
            pl.BlockSpec((_TCB,), lambda i: (i,)),
        ],
        out_shape=[
            jax.ShapeDtypeStruct((NP, D), jnp.float32),
            jax.ShapeDtypeStruct((NP,), jnp.float32),
            jax.ShapeDtypeStruct((NP,), jnp.float32),
        ],
    )(x_p, W, a_s, a_d)


def _normalized(acc_ref, den_ref, b_ref):
    a = jnp.sum(acc_ref[...], axis=0)
    dn = jnp.sum(den_ref[...].T, axis=1, keepdims=True)
    o = a / jnp.maximum(dn, 1e-16) + b_ref[...][None, :]
    return jnp.maximum(o, 0.0)


def _tc_mid_body(acc_ref, den_ref, b_ref, w_ref, as_ref, ad_ref,
                 h_ref, s_ref, d_ref):
    o = _normalized(acc_ref, den_ref, b_ref)
    h = jnp.dot(o, w_ref[...], preferred_element_type=jnp.float32)
    h_ref[...] = h
    s_ref[...] = jnp.sum(h * as_ref[...][None, :], axis=1)
    d_ref[...] = jnp.sum(h * ad_ref[...][None, :], axis=1)


def _tc_mid(acc, den_t, b, W, a_s, a_d):
    return pl.pallas_call(
        _tc_mid_body,
        grid=(NP // _TCB,),
        in_specs=[
            pl.BlockSpec((2, _TCB, D), lambda i: (0, i, 0)),
            pl.BlockSpec((NW, _TCB), lambda i: (0, i)),
            pl.BlockSpec((D,), lambda i: (0,)),
            pl.BlockSpec((D, D), lambda i: (0, 0)),
            pl.BlockSpec((D,), lambda i: (0,)),
            pl.BlockSpec((D,), lambda i: (0,)),
        ],
        out_specs=[
            pl.BlockSpec((_TCB, D), lambda i: (i, 0)),
            pl.BlockSpec((_TCB,), lambda i: (i,)),
            pl.BlockSpec((_TCB,), lambda i: (i,)),
        ],
        out_shape=[
            jax.ShapeDtypeStruct((NP, D), jnp.float32),
            jax.ShapeDtypeStruct((NP,), jnp.float32),
            jax.ShapeDtypeStruct((NP,), jnp.float32),
        ],
    )(acc, den_t, b, W, a_s, a_d)


def _tc_last_body(acc_ref, den_ref, b_ref, wf_ref, bf_ref, y_ref):
    o = _normalized(acc_ref, den_ref, b_ref)
    y_ref[...] = jnp.sum(o * wf_ref[...][None, :], axis=1) + bf_ref[0]


def _tc_last(acc, den_t, b, wf, bf):
    return pl.pallas_call(
        _tc_last_body,
        grid=(NP // _TCB,),
        in_specs=[
            pl.BlockSpec((2, _TCB, D), lambda i: (0, i, 0)),
            pl.BlockSpec((NW, _TCB), lambda i: (0, i)),
            pl.BlockSpec((D,), lambda i: (0,)),
            pl.BlockSpec((D,), lambda i: (0,)),
            pl.BlockSpec(memory_space=pltpu.SMEM),
        ],
        out_specs=pl.BlockSpec((_TCB,), lambda i: (i,)),
        out_shape=jax.ShapeDtypeStruct((NP,), jnp.float32),
    )(acc, den_t, b, wf, bf)


def _sc_edge_body(h_hbm, s_hbm, d_hbm, epk_hbm,
                  acc_out, den_out,
                  ep_v, src_v, dst_v, rows_v, ep_v2, src_v2, dst_v2,
                  val_v, asrc_v, adst_v, den_v,
                  acc_sh, s_sh, d_sh, sem, sem2):
    cid = lax.axis_index("c")
    sid = lax.axis_index("s")
    wid = sid * NC + cid
    base = wid * EPW

    zero16 = jnp.zeros((L,), jnp.float32)

    def zden(i, carry):
        den_v[pl.ds(i * L, L)] = zero16
        return carry
    lax.fori_loop(0, NP // L, zden, 0)

    def zrow(j, carry):
        for c8 in range(D // L):
            rows_v[j, pl.ds(c8 * L, L)] = zero16
        return carry
    lax.fori_loop(0, 2 * K, zrow, 0)
    for k in range(RPS // (2 * K)):
        pltpu.sync_copy(rows_v,
                        acc_sh.at[pl.ds(sid * RPS + k * 2 * K, 2 * K)])

    @pl.when(sid == 0)
    def _():
        pltpu.sync_copy(s_hbm, s_sh)
        pltpu.sync_copy(d_hbm, d_sh)
    plsc.subcore_barrier()
    pltpu.sync_copy(s_sh, asrc_v)
    pltpu.sync_copy(d_sh, adst_v)
    plsc.subcore_barrier()

    bufs = ((ep_v, src_v, dst_v, 0, sem),
            (ep_v2, src_v2, dst_v2, K, sem2))

    def start_gather(c, ev, sv, dv, ro, sm):
        off = pl.multiple_of(base + c * K, 8)
        pltpu.sync_copy(epk_hbm.at[pl.ds(off, K)], ev)

        def unpack(g, carry2):
            pk = ev[pl.ds(g * L, L)]
            sv[pl.ds(g * L, L)] = lax.shift_right_logical(pk, 14)
            dv[pl.ds(g * L, L)] = lax.bitwise_and(pk, 16383)
            return carry2
        lax.fori_loop(0, K // L, unpack, 0)
        pltpu.async_copy(h_hbm.at[sv], rows_v.at[pl.ds(ro, K)], sm)

    def vals_compute(sv, dv):
        def vals(j, carry2):
            si = sv[pl.ds(j * L, L)]
            di = dv[pl.ds(j * L, L)]
            e = plsc.load_gather(asrc_v, [si]) + plsc.load_gather(adst_v, [di])
            e = jnp.where(e >= 0.0, e, 0.2 * e)
            v = jnp.exp(e)
            val_v[pl.ds(j * L, L)] = v
            plsc.addupdate_scatter(den_v, [di], v)
            return carry2
        lax.fori_loop(0, K // L, vals, 0)

    def scale_scatter(ro, dv):
        def scale(g, carry2):
            vv = val_v[pl.ds(g * L, L)]
            for i in range(L):
                v = vv[i]
                j = ro + g * L + i
                for c8 in range(D // L):
                    rows_v[j, pl.ds(c8 * L, L)] = (
                        rows_v[j, pl.ds(c8 * L, L)] * v)
            return carry2
        lax.fori_loop(0, K // L, scale, 0)
        pltpu.sync_copy(rows_v.at[pl.ds(ro, K)], acc_sh.at[dv], add=True)

    def process(cur, p):
        ev, sv, dv, ro, sm = bufs[p]
        start_gather(cur + 1, *bufs[1 - p])
        vals_compute(sv, dv)
        pltpu.make_async_copy(h_hbm.at[sv], rows_v.at[pl.ds(ro, K)],
                              sm).wait()
        scale_scatter(ro, dv)

    start_gather(0, *bufs[0])

    def pair(k, carry):
        process(2 * k, 0)
        process(2 * k + 1, 1)
        return carry
    lax.fori_loop(0, (CH - 1) // 2, pair, 0)

    vals_compute(src_v, dst_v)
    pltpu.make_async_copy(h_hbm.at[src_v], rows_v.at[pl.ds(0, K)],
                          sem).wait()
    scale_scatter(0, dst_v)

    pltpu.sync_copy(den_v, den_out.at[cid, sid])
    plsc.subcore_barrier()
    pltpu.sync_copy(acc_sh.at[pl.ds(sid * RPS, RPS)],
                    acc_out.at[cid, pl.ds(sid * RPS, RPS)])


def _make_sc_edge():
    mesh = plsc.VectorSubcoreMesh(core_axis_name="c", subcore_axis_name="s")
    return pl.kernel(
        _sc_edge_body,
        out_type=[
            jax.ShapeDtypeStruct((2, NP, D), jnp.float32),
            jax.ShapeDtypeStruct((2, NS, NP), jnp.float32),
        ],
        mesh=mesh,
        scratch_types=[
            pltpu.VMEM((K,), jnp.int32),
            pltpu.VMEM((K,), jnp.int32),
            pltpu.VMEM((K,), jnp.int32),
            pltpu.VMEM((2 * K, D), jnp.float32),
            pltpu.VMEM((K,), jnp.int32),
            pltpu.VMEM((K,), jnp.int32),
            pltpu.VMEM((K,), jnp.int32),
            pltpu.VMEM((K,), jnp.float32),
            pltpu.VMEM((NP,), jnp.float32),
            pltpu.VMEM((NP,), jnp.float32),
            pltpu.VMEM((NP,), jnp.float32),
            pltpu.VMEM_SHARED((NP, D), jnp.float32),
            pltpu.VMEM_SHARED((NP,), jnp.float32),
            pltpu.VMEM_SHARED((NP,), jnp.float32),
            pltpu.SemaphoreType.DMA,
            pltpu.SemaphoreType.DMA,
        ],
        compiler_params=pltpu.CompilerParams(needs_layout_passes=False),
    )


_sc_edge = _make_sc_edge()


def kernel(x, edge_indices, W1, a_src1, a_dst1, b1,
           W2, a_src2, a_dst2, b2, Wf, bf):
    xs = jnp.squeeze(x, axis=0)
    x_p = jnp.pad(xs, ((0, NP - N), (0, 0)))
    ei = jnp.squeeze(edge_indices, axis=0)
    src = ei[:, 0].astype(jnp.int32)
    dst = ei[:, 1].astype(jnp.int32)
    epk = jnp.left_shift(src, 14) | dst

    def den_t(den):
        return den.reshape(NW, NP)

    h1, s1, d1 = _tc_first(x_p, W1, a_src1.reshape(D), a_dst1.reshape(D))
    acc1, den1 = _sc_edge(h1, s1, d1, epk)
    h2, s2, d2 = _tc_mid(acc1, den_t(den1), b1, W2,
                         a_src2.reshape(D), a_dst2.reshape(D))
    acc2, den2 = _sc_edge(h2, s2, d2, epk)
    y = _tc_last(acc2, den_t(den2), b2, Wf.reshape(D), bf)
    return y[:N]


# speedup vs baseline: 1.9166x; 1.9166x over previous
import functools

import jax
import jax.numpy as jnp
from jax import lax
from jax.experimental import pallas as pl
from jax.experimental.pallas import tpu as pltpu
from jax.experimental.pallas import tpu_sc as plsc

N = 10000
NP = 10240
D = 128
E = 320000
NC = 2
NS = 16
NW = NC * NS
L = 16
EPW = E // NW
K = 64
CH = EPW // K
TAIL = EPW - CH * K
RPS = NP // NS


_TCB = 2048


def _tc_first_body(x_ref, w_ref, as_ref, ad_ref, h_ref, s_ref, d_ref):
    h = jnp.dot(x_ref[...], w_ref[...], preferred_element_type=jnp.float32)
    h_ref[...] = h
    s_ref[...] = jnp.sum(h * as_ref[...][None, :], axis=1)
    d_ref[...] = jnp.sum(h * ad_ref[...][None, :], axis=1)


def _tc_first(x_p, W, a_s, a_d):
    return pl.pallas_call(
        _tc_first_body,
        grid=(NP // _TCB,),
        in_specs=[
            pl.BlockSpec((_TCB, D), lambda i: (i, 0)),
            pl.BlockSpec((D, D), lambda i: (0, 0)),
            pl.BlockSpec((D,), lambda i: (0,)),
            pl.BlockSpec((D,), lambda i: (0,)),
        ],
        out_specs=[
            pl.BlockSpec((_TCB, D), lambda i: (i, 0)),
            pl.BlockSpec((_TCB,), lambda i: (i,)),
            pl.BlockSpec((_TCB,), lambda i: (i,)),
        ],
        out_shape=[
            jax.ShapeDtypeStruct((NP, D), jnp.float32),
            jax.ShapeDtypeStruct((NP,), jnp.float32),
            jax.ShapeDtypeStruct((NP,), jnp.float32),
        ],
    )(x_p, W, a_s, a_d)


def _normalized(acc_ref, den_ref, b_ref):
    a = jnp.sum(acc_ref[...], axis=0)
    dn = jnp.sum(den_ref[...].T, axis=1, keepdims=True)
    o = a / jnp.maximum(dn, 1e-16) + b_ref[...][None, :]
    return jnp.maximum(o, 0.0)


def _tc_mid_body(acc_ref, den_ref, b_ref, w_ref, as_ref, ad_ref,
                 h_ref, s_ref, d_ref):
    o = _normalized(acc_ref, den_ref, b_ref)
    h = jnp.dot(o, w_ref[...], preferred_element_type=jnp.float32)
    h_ref[...] = h
    s_ref[...] = jnp.sum(h * as_ref[...][None, :], axis=1)
    d_ref[...] = jnp.sum(h * ad_ref[...][None, :], axis=1)


def _tc_mid(acc, den_t, b, W, a_s, a_d):
    return pl.pallas_call(
        _tc_mid_body,
        grid=(NP // _TCB,),
        in_specs=[
            pl.BlockSpec((2, _TCB, D), lambda i: (0, i, 0)),
            pl.BlockSpec((NW, _TCB), lambda i: (0, i)),
            pl.BlockSpec((D,), lambda i: (0,)),
            pl.BlockSpec((D, D), lambda i: (0, 0)),
            pl.BlockSpec((D,), lambda i: (0,)),
            pl.BlockSpec((D,), lambda i: (0,)),
        ],
        out_specs=[
            pl.BlockSpec((_TCB, D), lambda i: (i, 0)),
            pl.BlockSpec((_TCB,), lambda i: (i,)),
            pl.BlockSpec((_TCB,), lambda i: (i,)),
        ],
        out_shape=[
            jax.ShapeDtypeStruct((NP, D), jnp.float32),
            jax.ShapeDtypeStruct((NP,), jnp.float32),
            jax.ShapeDtypeStruct((NP,), jnp.float32),
        ],
    )(acc, den_t, b, W, a_s, a_d)


def _tc_last_body(acc_ref, den_ref, b_ref, wf_ref, bf_ref, y_ref):
    o = _normalized(acc_ref, den_ref, b_ref)
    y_ref[...] = jnp.sum(o * wf_ref[...][None, :], axis=1) + bf_ref[0]


def _tc_last(acc, den_t, b, wf, bf):
    return pl.pallas_call(
        _tc_last_body,
        grid=(NP // _TCB,),
        in_specs=[
            pl.BlockSpec((2, _TCB, D), lambda i: (0, i, 0)),
            pl.BlockSpec((NW, _TCB), lambda i: (0, i)),
            pl.BlockSpec((D,), lambda i: (0,)),
            pl.BlockSpec((D,), lambda i: (0,)),
            pl.BlockSpec(memory_space=pltpu.SMEM),
        ],
        out_specs=pl.BlockSpec((_TCB,), lambda i: (i,)),
        out_shape=jax.ShapeDtypeStruct((NP,), jnp.float32),
    )(acc, den_t, b, wf, bf)


def _sc_edge_body(h_hbm, s_hbm, d_hbm, epk_hbm,
                  acc_out, den_out,
                  ep_v, src_v, dst_v, rows_v, ep_v2, src_v2, dst_v2,
                  ep_t, src_t, dst_t,
                  val_v, asrc_v, adst_v, den_v,
                  acc_sh, sem, sem2):
    cid = lax.axis_index("c")
    sid = lax.axis_index("s")
    wid = sid * NC + cid
    base = wid * EPW

    zero16 = jnp.zeros((L,), jnp.float32)

    def zden(i, carry):
        den_v[pl.ds(i * L, L)] = zero16
        return carry
    lax.fori_loop(0, NP // L, zden, 0)

    def zrow(j, carry):
        for c8 in range(D // L):
            rows_v[j, pl.ds(c8 * L, L)] = zero16
        return carry
    lax.fori_loop(0, 2 * K, zrow, 0)
    for k in range(RPS // (2 * K)):
        pltpu.sync_copy(rows_v,
                        acc_sh.at[pl.ds(sid * RPS + k * 2 * K, 2 * K)])

    pltpu.sync_copy(s_hbm, asrc_v)
    pltpu.sync_copy(d_hbm, adst_v)
    plsc.subcore_barrier()

    bufs = ((ep_v, src_v, dst_v, 0, sem),
            (ep_v2, src_v2, dst_v2, K, sem2))

    def start_gather(c, ev, sv, dv, ro, sm):
        off = pl.multiple_of(base + c * K, 8)
        pltpu.sync_copy(epk_hbm.at[pl.ds(off, K)], ev)

        def unpack(g, carry2):
            pk = ev[pl.ds(g * L, L)]
            sv[pl.ds(g * L, L)] = lax.shift_right_logical(pk, 14)
            dv[pl.ds(g * L, L)] = lax.bitwise_and(pk, 16383)
            return carry2
        lax.fori_loop(0, K // L, unpack, 0)
        pltpu.async_copy(h_hbm.at[sv], rows_v.at[pl.ds(ro, K)], sm)

    def vals_compute(sv, dv):
        def vals(j, carry2):
            si = sv[pl.ds(j * L, L)]
            di = dv[pl.ds(j * L, L)]
            e = plsc.load_gather(asrc_v, [si]) + plsc.load_gather(adst_v, [di])
            e = jnp.where(e >= 0.0, e, 0.2 * e)
            v = jnp.exp(e)
            val_v[pl.ds(j * L, L)] = v
            plsc.addupdate_scatter(den_v, [di], v)
            return carry2
        lax.fori_loop(0, K // L, vals, 0)

    def scale_scatter(ro, dv):
        def scale(g, carry2):
            vv = val_v[pl.ds(g * L, L)]
            for i in range(L):
                v = vv[i]
                j = ro + g * L + i
                for c8 in range(D // L):
                    rows_v[j, pl.ds(c8 * L, L)] = (
                        rows_v[j, pl.ds(c8 * L, L)] * v)
            return carry2
        lax.fori_loop(0, K // L, scale, 0)
        pltpu.sync_copy(rows_v.at[pl.ds(ro, K)], acc_sh.at[dv], add=True)

    def process(cur, p):
        ev, sv, dv, ro, sm = bufs[p]
        start_gather(cur + 1, *bufs[1 - p])
        vals_compute(sv, dv)
        pltpu.make_async_copy(h_hbm.at[sv], rows_v.at[pl.ds(ro, K)],
                              sm).wait()
        scale_scatter(ro, dv)

    start_gather(0, *bufs[0])

    def pair(k, carry):
        process(2 * k, 0)
        process(2 * k + 1, 1)
        return carry
    lax.fori_loop(0, (CH - 2) // 2, pair, 0)

    process(CH - 2, 0)
    vals_compute(src_v2, dst_v2)
    pltpu.make_async_copy(h_hbm.at[src_v2], rows_v.at[pl.ds(K, K)],
                          sem2).wait()
    scale_scatter(K, dst_v2)

    pltpu.sync_copy(epk_hbm.at[pl.ds(base + CH * K, TAIL)], ep_t)
    pkt = ep_t[...]
    src_t[...] = lax.shift_right_logical(pkt, 14)
    dst_t[...] = lax.bitwise_and(pkt, 16383)
    pltpu.async_copy(h_hbm.at[src_t], rows_v.at[pl.ds(0, TAIL)], sem).wait()
    sit = src_t[...]
    dit = dst_t[...]
    et = plsc.load_gather(asrc_v, [sit]) + plsc.load_gather(adst_v, [dit])
    et = jnp.where(et >= 0.0, et, 0.2 * et)
    vt = jnp.exp(et)
    plsc.addupdate_scatter(den_v, [dit], vt)
    for i in range(L):
        vti = vt[i]
        for c8 in range(D // L):
            rows_v[i, pl.ds(c8 * L, L)] = rows_v[i, pl.ds(c8 * L, L)] * vti
    pltpu.sync_copy(rows_v.at[pl.ds(0, TAIL)], acc_sh.at[dst_t], add=True)

    pltpu.sync_copy(den_v, den_out.at[cid, sid])
    plsc.subcore_barrier()
    pltpu.sync_copy(acc_sh.at[pl.ds(sid * RPS, RPS)],
                    acc_out.at[cid, pl.ds(sid * RPS, RPS)])


def _make_sc_edge():
    mesh = plsc.VectorSubcoreMesh(core_axis_name="c", subcore_axis_name="s")
    return pl.kernel(
        _sc_edge_body,
        out_type=[
            jax.ShapeDtypeStruct((2, NP, D), jnp.float32),
            jax.ShapeDtypeStruct((2, NS, NP), jnp.float32),
        ],
        mesh=mesh,
        scratch_types=[
            pltpu.VMEM((K,), jnp.int32),
            pltpu.VMEM((K,), jnp.int32),
            pltpu.VMEM((K,), jnp.int32),
            pltpu.VMEM((2 * K, D), jnp.float32),
            pltpu.VMEM((K,), jnp.int32),
            pltpu.VMEM((K,), jnp.int32),
            pltpu.VMEM((K,), jnp.int32),
            pltpu.VMEM((TAIL,), jnp.int32),
            pltpu.VMEM((TAIL,), jnp.int32),
            pltpu.VMEM((TAIL,), jnp.int32),
            pltpu.VMEM((K,), jnp.float32),
            pltpu.VMEM((NP,), jnp.float32),
            pltpu.VMEM((NP,), jnp.float32),
            pltpu.VMEM((NP,), jnp.float32),
            pltpu.VMEM_SHARED((NP, D), jnp.float32),
            pltpu.SemaphoreType.DMA,
            pltpu.SemaphoreType.DMA,
        ],
        compiler_params=pltpu.CompilerParams(needs_layout_passes=False),
    )


_sc_edge = _make_sc_edge()


def kernel(x, edge_indices, W1, a_src1, a_dst1, b1,
           W2, a_src2, a_dst2, b2, Wf, bf):
    xs = jnp.squeeze(x, axis=0)
    x_p = jnp.pad(xs, ((0, NP - N), (0, 0)))
    ei = jnp.squeeze(edge_indices, axis=0)
    src = ei[:, 0].astype(jnp.int32)
    dst = ei[:, 1].astype(jnp.int32)
    epk = jnp.left_shift(src, 14) | dst

    def den_t(den):
        return den.reshape(NW, NP)

    h1, s1, d1 = _tc_first(x_p, W1, a_src1.reshape(D), a_dst1.reshape(D))
    acc1, den1 = _sc_edge(h1, s1, d1, epk)
    h2, s2, d2 = _tc_mid(acc1, den_t(den1), b1, W2,
                         a_src2.reshape(D), a_dst2.reshape(D))
    acc2, den2 = _sc_edge(h2, s2, d2, epk)
    y = _tc_last(acc2, den_t(den2), b2, Wf.reshape(D), bf)
    return y[:N]
